# Initial kernel scaffold; baseline (speedup 1.0000x reference)
import jax
import jax.numpy as jnp
from jax.experimental import pallas as pl


def kernel(x_0, t, clause_mask, Q_bar_mats):
    raise NotImplementedError("write your pallas kernel here")


# speedup vs baseline: 2.1573x; 2.1573x over previous
import jax
import jax.numpy as jnp
from jax.experimental import pallas as pl
from jax.experimental.pallas import tpu as pltpu

_NUM_CLASSES = 3
_B, _N, _M = 32, 256, 1024
_N_CHUNK = 128

_KS0 = 0
_KS1 = 12345
_KS2 = _KS0 ^ _KS1 ^ 0x1BD11BDA

_ROT0 = (13, 15, 26, 6)
_ROT1 = (17, 29, 16, 24)


def _rotl(x, d):
    return (x << jnp.uint32(d)) | (x >> jnp.uint32(32 - d))


def _four_rounds(x0, x1, rots):
    for r in rots:
        x0 = x0 + x1
        x1 = _rotl(x1, r)
        x1 = x1 ^ x0
    return x0, x1


def _threefry_xored(j):
    ks0 = jnp.uint32(_KS0)
    ks1 = jnp.uint32(_KS1)
    ks2 = jnp.uint32(_KS2)
    x0 = jnp.zeros_like(j) + ks0
    x1 = j + ks1
    x0, x1 = _four_rounds(x0, x1, _ROT0)
    x0 = x0 + ks1
    x1 = x1 + (ks2 + jnp.uint32(1))
    x0, x1 = _four_rounds(x0, x1, _ROT1)
    x0 = x0 + ks2
    x1 = x1 + (ks0 + jnp.uint32(2))
    x0, x1 = _four_rounds(x0, x1, _ROT0)
    x0 = x0 + ks0
    x1 = x1 + (ks1 + jnp.uint32(3))
    x0, x1 = _four_rounds(x0, x1, _ROT1)
    x0 = x0 + ks1
    x1 = x1 + (ks2 + jnp.uint32(4))
    x0, x1 = _four_rounds(x0, x1, _ROT0)
    x0 = x0 + ks2
    x1 = x1 + (ks0 + jnp.uint32(5))
    return x0 ^ x1


def _gumbel_from_bits(bits):
    tiny = jnp.float32(jnp.finfo(jnp.float32).tiny)
    one = jnp.float32(1.0)
    float_bits = (bits >> jnp.uint32(9)) | jnp.uint32(0x3F800000)
    floats = jax.lax.bitcast_convert_type(float_bits, jnp.float32) - one
    u = jnp.maximum(tiny, floats * (one - tiny) + tiny)
    return -jnp.log(-jnp.log(u))


def _body(t_ref, x0_ref, mask_ref, qbar_ref, out_ref):
    b = pl.program_id(0)
    nc = pl.program_id(1)

    x = x0_ref[0]
    shape = x.shape

    qrow = qbar_ref[0].astype(jnp.bfloat16).astype(jnp.float32)
    logq = jnp.log(jnp.clip(qrow, 1e-20, None))

    def bcast(row, col):
        sl = jax.lax.slice(logq, (0, 3 * row + col), (1, 3 * row + col + 1))
        return jax.lax.broadcast_in_dim(sl, shape, (0, 1))

    ni = jax.lax.broadcasted_iota(jnp.uint32, shape, 0)
    mi = jax.lax.broadcasted_iota(jnp.uint32, shape, 1)
    n0 = (nc * _N_CHUNK).astype(jnp.uint32)
    cell = (b.astype(jnp.uint32) * jnp.uint32(_N) + n0 + ni) * jnp.uint32(_M) + mi
    j0 = cell * jnp.uint32(3)

    is1 = x == 1
    is2 = x == 2

    v = []
    for c in range(_NUM_CLASSES):
        g = _gumbel_from_bits(_threefry_xored(j0 + jnp.uint32(c)))
        lc = jnp.where(is2, bcast(2, c), jnp.where(is1, bcast(1, c), bcast(0, c)))
        v.append(g + lc)

    idx = jnp.where(v[1] > v[0], 1, 0).astype(jnp.int32)
    vm = jnp.maximum(v[0], v[1])
    res = jnp.where(v[2] > vm, 2, idx).astype(jnp.int32)

    m = mask_ref[0]
    out_ref[0] = jnp.where(m != 0, res, 0)


@jax.jit
def kernel(x_0, t, clause_mask, Q_bar_mats):
    x_0 = x_0.astype(jnp.int32)
    t = t.astype(jnp.int32)
    mask = clause_mask.astype(jnp.int32).reshape(_B, 1, _M)
    qbar = Q_bar_mats.astype(jnp.float32).reshape(1000, 1, 9)

    n_chunks = _N // _N_CHUNK
    grid = (_B, n_chunks)

    out = pl.pallas_call(
        _body,
        grid_spec=pltpu.PrefetchScalarGridSpec(
            num_scalar_prefetch=1,
            grid=grid,
            in_specs=[
                pl.BlockSpec((1, _N_CHUNK, _M), lambda b, nc, t_ref: (b, nc, 0)),
                pl.BlockSpec((1, 1, _M), lambda b, nc, t_ref: (b, 0, 0)),
                pl.BlockSpec((1, 1, 9), lambda b, nc, t_ref: (t_ref[b], 0, 0)),
            ],
            out_specs=pl.BlockSpec((1, _N_CHUNK, _M), lambda b, nc, t_ref: (b, nc, 0)),
        ),
        out_shape=jax.ShapeDtypeStruct((_B, _N, _M), jnp.int32),
    )(t, x_0, mask, qbar)
    return out


# speedup vs baseline: 2.1574x; 1.0001x over previous
import jax
import jax.numpy as jnp
from jax.experimental import pallas as pl
from jax.experimental.pallas import tpu as pltpu

_NUM_CLASSES = 3
_B, _N, _M = 32, 256, 1024
_N_CHUNK = 128

_KS0 = 0
_KS1 = 12345
_KS2 = _KS0 ^ _KS1 ^ 0x1BD11BDA

_ROT0 = (13, 15, 26, 6)
_ROT1 = (17, 29, 16, 24)


def _rotl(x, d):
    return (x << jnp.uint32(d)) | (x >> jnp.uint32(32 - d))


def _four_rounds(x0, x1, rots):
    for r in rots:
        x0 = x0 + x1
        x1 = _rotl(x1, r)
        x1 = x1 ^ x0
    return x0, x1


def _threefry_xored(j):
    ks0 = jnp.uint32(_KS0)
    ks1 = jnp.uint32(_KS1)
    ks2 = jnp.uint32(_KS2)
    x0 = jnp.zeros_like(j) + ks0
    x1 = j + ks1
    x0, x1 = _four_rounds(x0, x1, _ROT0)
    x0 = x0 + ks1
    x1 = x1 + (ks2 + jnp.uint32(1))
    x0, x1 = _four_rounds(x0, x1, _ROT1)
    x0 = x0 + ks2
    x1 = x1 + (ks0 + jnp.uint32(2))
    x0, x1 = _four_rounds(x0, x1, _ROT0)
    x0 = x0 + ks0
    x1 = x1 + (ks1 + jnp.uint32(3))
    x0, x1 = _four_rounds(x0, x1, _ROT1)
    x0 = x0 + ks1
    x1 = x1 + (ks2 + jnp.uint32(4))
    x0, x1 = _four_rounds(x0, x1, _ROT0)
    x0 = x0 + ks2
    x1 = x1 + (ks0 + jnp.uint32(5))
    return x0 ^ x1


def _gumbel_from_bits(bits):
    tiny = jnp.float32(jnp.finfo(jnp.float32).tiny)
    one = jnp.float32(1.0)
    float_bits = (bits >> jnp.uint32(9)) | jnp.uint32(0x3F800000)
    floats = jax.lax.bitcast_convert_type(float_bits, jnp.float32) - one
    u = jnp.maximum(tiny, floats * (one - tiny) + tiny)
    return -jnp.log(-jnp.log(u))


def _body(t_ref, x0_ref, mask_ref, qbar_ref, out_ref):
    b = pl.program_id(0)
    nc = pl.program_id(1)

    x = x0_ref[0]
    shape = x.shape

    qrow = qbar_ref[0].astype(jnp.bfloat16).astype(jnp.float32)
    logq = jnp.log(jnp.clip(qrow, 1e-20, None))

    def bcast(row, col):
        sl = jax.lax.slice(logq, (0, 3 * row + col), (1, 3 * row + col + 1))
        return jax.lax.broadcast_in_dim(sl, shape, (0, 1))

    ni = jax.lax.broadcasted_iota(jnp.uint32, shape, 0)
    mi = jax.lax.broadcasted_iota(jnp.uint32, shape, 1)
    n0 = (nc * _N_CHUNK).astype(jnp.uint32)
    cell = (b.astype(jnp.uint32) * jnp.uint32(_N) + n0 + ni) * jnp.uint32(_M) + mi
    j0 = cell * jnp.uint32(3)

    is1 = x == 1
    is2 = x == 2

    v = []
    for c in range(_NUM_CLASSES):
        g = _gumbel_from_bits(_threefry_xored(j0 + jnp.uint32(c)))
        lc = jnp.where(is2, bcast(2, c), jnp.where(is1, bcast(1, c), bcast(0, c)))
        v.append(g + lc)

    idx = jnp.where(v[1] > v[0], 1, 0).astype(jnp.int32)
    vm = jnp.maximum(v[0], v[1])
    res = jnp.where(v[2] > vm, 2, idx).astype(jnp.int32)

    m = mask_ref[0]
    out_ref[0] = jnp.where(m != 0, res, 0)


@jax.jit
def kernel(x_0, t, clause_mask, Q_bar_mats):
    x_0 = x_0.astype(jnp.int32)
    t = t.astype(jnp.int32)
    mask = clause_mask.astype(jnp.int32).reshape(_B, 1, _M)
    qbar = Q_bar_mats.astype(jnp.float32).reshape(1000, 1, 9)

    n_chunks = _N // _N_CHUNK
    grid = (_B, n_chunks)

    out = pl.pallas_call(
        _body,
        grid_spec=pltpu.PrefetchScalarGridSpec(
            num_scalar_prefetch=1,
            grid=grid,
            in_specs=[
                pl.BlockSpec((1, _N_CHUNK, _M), lambda b, nc, t_ref: (b, nc, 0)),
                pl.BlockSpec((1, 1, _M), lambda b, nc, t_ref: (b, 0, 0)),
                pl.BlockSpec((1, 1, 9), lambda b, nc, t_ref: (t_ref[b], 0, 0)),
            ],
            out_specs=pl.BlockSpec((1, _N_CHUNK, _M), lambda b, nc, t_ref: (b, nc, 0)),
        ),
        out_shape=jax.ShapeDtypeStruct((_B, _N, _M), jnp.int32),
        compiler_params=pltpu.CompilerParams(
            dimension_semantics=("parallel", "parallel"),
        ),
    )(t, x_0, mask, qbar)
    return out


# speedup vs baseline: 2.1945x; 1.0172x over previous
import jax
import jax.numpy as jnp
from jax.experimental import pallas as pl
from jax.experimental.pallas import tpu as pltpu

_NUM_CLASSES = 3
_B, _N, _M = 32, 256, 1024
_N_CHUNK = 128

_KS0 = 0
_KS1 = 12345
_KS2 = _KS0 ^ _KS1 ^ 0x1BD11BDA

_ROT0 = (13, 15, 26, 6)
_ROT1 = (17, 29, 16, 24)


def _rotl(x, d):
    return (x << jnp.uint32(d)) | (x >> jnp.uint32(32 - d))


def _four_rounds(x0, x1, rots):
    for r in rots:
        x0 = x0 + x1
        x1 = _rotl(x1, r)
        x1 = x1 ^ x0
    return x0, x1


def _threefry_xored(j):
    ks0 = jnp.uint32(_KS0)
    ks1 = jnp.uint32(_KS1)
    ks2 = jnp.uint32(_KS2)
    x0 = jnp.zeros_like(j) + ks0
    x1 = j + ks1
    x0, x1 = _four_rounds(x0, x1, _ROT0)
    x0 = x0 + ks1
    x1 = x1 + (ks2 + jnp.uint32(1))
    x0, x1 = _four_rounds(x0, x1, _ROT1)
    x0 = x0 + ks2
    x1 = x1 + (ks0 + jnp.uint32(2))
    x0, x1 = _four_rounds(x0, x1, _ROT0)
    x0 = x0 + ks0
    x1 = x1 + (ks1 + jnp.uint32(3))
    x0, x1 = _four_rounds(x0, x1, _ROT1)
    x0 = x0 + ks1
    x1 = x1 + (ks2 + jnp.uint32(4))
    x0, x1 = _four_rounds(x0, x1, _ROT0)
    x0 = x0 + ks2
    x1 = x1 + (ks0 + jnp.uint32(5))
    return x0 ^ x1


def _neglog_u_from_bits(bits):
    tiny = jnp.float32(jnp.finfo(jnp.float32).tiny)
    one = jnp.float32(1.0)
    float_bits = (bits >> jnp.uint32(9)) | jnp.uint32(0x3F800000)
    floats = jax.lax.bitcast_convert_type(float_bits, jnp.float32) - one
    u = jnp.maximum(tiny, floats * (one - tiny) + tiny)
    return -jnp.log(u)


def _body(t_ref, x0_ref, mask_ref, qbar_ref, out_ref):
    b = pl.program_id(0)
    nc = pl.program_id(1)

    x = x0_ref[0]
    shape = x.shape

    qrow = qbar_ref[0].astype(jnp.bfloat16).astype(jnp.float32)
    winv = 1.0 / jnp.clip(qrow, 1e-20, None)

    def bcast(row, col):
        sl = jax.lax.slice(winv, (0, 3 * row + col), (1, 3 * row + col + 1))
        return jax.lax.broadcast_in_dim(sl, shape, (0, 1))

    ni = jax.lax.broadcasted_iota(jnp.uint32, shape, 0)
    mi = jax.lax.broadcasted_iota(jnp.uint32, shape, 1)
    n0 = (nc * _N_CHUNK).astype(jnp.uint32)
    cell = (b.astype(jnp.uint32) * jnp.uint32(_N) + n0 + ni) * jnp.uint32(_M) + mi
    j0 = cell * jnp.uint32(3)

    is1 = x == 1
    is2 = x == 2

    s = []
    for c in range(_NUM_CLASSES):
        e = _neglog_u_from_bits(_threefry_xored(j0 + jnp.uint32(c)))
        wc = jnp.where(is2, bcast(2, c), jnp.where(is1, bcast(1, c), bcast(0, c)))
        s.append(e * wc)

    idx = jnp.where(s[1] < s[0], 1, 0).astype(jnp.int32)
    sm = jnp.minimum(s[0], s[1])
    res = jnp.where(s[2] < sm, 2, idx).astype(jnp.int32)

    m = mask_ref[0]
    out_ref[0] = jnp.where(m != 0, res, 0)


@jax.jit
def kernel(x_0, t, clause_mask, Q_bar_mats):
    x_0 = x_0.astype(jnp.int32)
    t = t.astype(jnp.int32)
    mask = clause_mask.astype(jnp.int32).reshape(_B, 1, _M)
    qbar = Q_bar_mats.astype(jnp.float32).reshape(1000, 1, 9)

    n_chunks = _N // _N_CHUNK
    grid = (_B, n_chunks)

    out = pl.pallas_call(
        _body,
        grid_spec=pltpu.PrefetchScalarGridSpec(
            num_scalar_prefetch=1,
            grid=grid,
            in_specs=[
                pl.BlockSpec((1, _N_CHUNK, _M), lambda b, nc, t_ref: (b, nc, 0)),
                pl.BlockSpec((1, 1, _M), lambda b, nc, t_ref: (b, 0, 0)),
                pl.BlockSpec((1, 1, 9), lambda b, nc, t_ref: (t_ref[b], 0, 0)),
            ],
            out_specs=pl.BlockSpec((1, _N_CHUNK, _M), lambda b, nc, t_ref: (b, nc, 0)),
        ),
        out_shape=jax.ShapeDtypeStruct((_B, _N, _M), jnp.int32),
        compiler_params=pltpu.CompilerParams(
            dimension_semantics=("parallel", "parallel"),
        ),
    )(t, x_0, mask, qbar)
    return out


# speedup vs baseline: 3.4386x; 1.5669x over previous
import jax
import jax.numpy as jnp
from jax.experimental import pallas as pl
from jax.experimental.pallas import tpu as pltpu

_NUM_CLASSES = 3
_B, _N, _M = 32, 256, 1024
_N_CHUNK = 128

_KS0 = 0
_KS1 = 12345
_KS2 = _KS0 ^ _KS1 ^ 0x1BD11BDA

_ROT0 = (13, 15, 26, 6)
_ROT1 = (17, 29, 16, 24)


def _rotl(x, d):
    return (x << jnp.uint32(d)) | (x >> jnp.uint32(32 - d))


def _four_rounds(x0, x1, rots):
    for r in rots:
        x0 = x0 + x1
        x1 = _rotl(x1, r)
        x1 = x1 ^ x0
    return x0, x1


def _threefry_xored(j):
    ks0 = jnp.uint32(_KS0)
    ks1 = jnp.uint32(_KS1)
    ks2 = jnp.uint32(_KS2)
    x0 = jnp.zeros_like(j) + ks0
    x1 = j + ks1
    x0, x1 = _four_rounds(x0, x1, _ROT0)
    x0 = x0 + ks1
    x1 = x1 + (ks2 + jnp.uint32(1))
    x0, x1 = _four_rounds(x0, x1, _ROT1)
    x0 = x0 + ks2
    x1 = x1 + (ks0 + jnp.uint32(2))
    x0, x1 = _four_rounds(x0, x1, _ROT0)
    x0 = x0 + ks0
    x1 = x1 + (ks1 + jnp.uint32(3))
    x0, x1 = _four_rounds(x0, x1, _ROT1)
    x0 = x0 + ks1
    x1 = x1 + (ks2 + jnp.uint32(4))
    x0, x1 = _four_rounds(x0, x1, _ROT0)
    x0 = x0 + ks2
    x1 = x1 + (ks0 + jnp.uint32(5))
    return x0 ^ x1


def _neglog_u_from_bits(bits):
    tiny = jnp.float32(jnp.finfo(jnp.float32).tiny)
    one = jnp.float32(1.0)
    float_bits = (bits >> jnp.uint32(9)) | jnp.uint32(0x3F800000)
    floats = jax.lax.bitcast_convert_type(float_bits, jnp.float32) - one
    u = jnp.maximum(tiny, floats * (one - tiny) + tiny)
    return -jnp.log(u)


_CH = 8


def _body(t_ref, x0_ref, mask_ref, qbar_ref, out_ref):
    b = pl.program_id(0)
    nc = pl.program_id(1)

    qrow = qbar_ref[0].astype(jnp.bfloat16).astype(jnp.float32)
    winv = 1.0 / jnp.clip(qrow, 1e-20, None)
    wsl = [jax.lax.slice(winv, (0, i), (1, i + 1)) for i in range(9)]

    m = mask_ref[0]

    shape = (_CH, _M)
    ni = jax.lax.broadcasted_iota(jnp.uint32, shape, 0)
    mi = jax.lax.broadcasted_iota(jnp.uint32, shape, 1)
    n0 = (nc * _N_CHUNK).astype(jnp.uint32)
    row0 = b.astype(jnp.uint32) * jnp.uint32(_N) + n0

    def bcast(row, col):
        return jax.lax.broadcast_in_dim(wsl[3 * row + col], shape, (0, 1))

    def chunk(i, carry):
        x = x0_ref[0, pl.ds(i * _CH, _CH), :]

        cell = (row0 + i.astype(jnp.uint32) * jnp.uint32(_CH) + ni) * jnp.uint32(_M) + mi
        j0 = cell * jnp.uint32(3)

        is1 = x == 1
        is2 = x == 2

        s = []
        for c in range(_NUM_CLASSES):
            e = _neglog_u_from_bits(_threefry_xored(j0 + jnp.uint32(c)))
            wc = jnp.where(is2, bcast(2, c), jnp.where(is1, bcast(1, c), bcast(0, c)))
            s.append(e * wc)

        idx = jnp.where(s[1] < s[0], 1, 0).astype(jnp.int32)
        sm = jnp.minimum(s[0], s[1])
        res = jnp.where(s[2] < sm, 2, idx).astype(jnp.int32)

        out_ref[0, pl.ds(i * _CH, _CH), :] = jnp.where(m != 0, res, 0)
        return carry

    jax.lax.fori_loop(0, _N_CHUNK // _CH, chunk, 0)


@jax.jit
def kernel(x_0, t, clause_mask, Q_bar_mats):
    x_0 = x_0.astype(jnp.int32)
    t = t.astype(jnp.int32)
    mask = clause_mask.astype(jnp.int32).reshape(_B, 1, _M)
    qbar = Q_bar_mats.astype(jnp.float32).reshape(1000, 1, 9)

    n_chunks = _N // _N_CHUNK
    grid = (_B, n_chunks)

    out = pl.pallas_call(
        _body,
        grid_spec=pltpu.PrefetchScalarGridSpec(
            num_scalar_prefetch=1,
            grid=grid,
            in_specs=[
                pl.BlockSpec((1, _N_CHUNK, _M), lambda b, nc, t_ref: (b, nc, 0)),
                pl.BlockSpec((1, 1, _M), lambda b, nc, t_ref: (b, 0, 0)),
                pl.BlockSpec((1, 1, 9), lambda b, nc, t_ref: (t_ref[b], 0, 0)),
            ],
            out_specs=pl.BlockSpec((1, _N_CHUNK, _M), lambda b, nc, t_ref: (b, nc, 0)),
        ),
        out_shape=jax.ShapeDtypeStruct((_B, _N, _M), jnp.int32),
        compiler_params=pltpu.CompilerParams(
            dimension_semantics=("parallel", "parallel"),
        ),
    )(t, x_0, mask, qbar)
    return out


# speedup vs baseline: 3.4551x; 1.0048x over previous
import jax
import jax.numpy as jnp
from jax.experimental import pallas as pl
from jax.experimental.pallas import tpu as pltpu

_NUM_CLASSES = 3
_B, _N, _M = 32, 256, 1024
_N_CHUNK = 128

_KS0 = 0
_KS1 = 12345
_KS2 = _KS0 ^ _KS1 ^ 0x1BD11BDA

_ROT0 = (13, 15, 26, 6)
_ROT1 = (17, 29, 16, 24)


def _rotl(x, d):
    return (x << jnp.uint32(d)) | (x >> jnp.uint32(32 - d))


def _four_rounds(x0, x1, rots):
    for r in rots:
        x0 = x0 + x1
        x1 = _rotl(x1, r)
        x1 = x1 ^ x0
    return x0, x1


def _threefry_xored(j):
    ks0 = jnp.uint32(_KS0)
    ks1 = jnp.uint32(_KS1)
    ks2 = jnp.uint32(_KS2)
    x0 = jnp.zeros_like(j) + ks0
    x1 = j + ks1
    x0, x1 = _four_rounds(x0, x1, _ROT0)
    x0 = x0 + ks1
    x1 = x1 + (ks2 + jnp.uint32(1))
    x0, x1 = _four_rounds(x0, x1, _ROT1)
    x0 = x0 + ks2
    x1 = x1 + (ks0 + jnp.uint32(2))
    x0, x1 = _four_rounds(x0, x1, _ROT0)
    x0 = x0 + ks0
    x1 = x1 + (ks1 + jnp.uint32(3))
    x0, x1 = _four_rounds(x0, x1, _ROT1)
    x0 = x0 + ks1
    x1 = x1 + (ks2 + jnp.uint32(4))
    x0, x1 = _four_rounds(x0, x1, _ROT0)
    x0 = x0 + ks2
    x1 = x1 + (ks0 + jnp.uint32(5))
    return x0 ^ x1


def _neglog_u_from_bits(bits):
    tiny = jnp.float32(jnp.finfo(jnp.float32).tiny)
    one = jnp.float32(1.0)
    float_bits = (bits >> jnp.uint32(9)) | jnp.uint32(0x3F800000)
    floats = jax.lax.bitcast_convert_type(float_bits, jnp.float32) - one
    u = jnp.maximum(tiny, floats * (one - tiny) + tiny)
    return -jnp.log(u)


_CH = 16


def _body(t_ref, x0_ref, mask_ref, qbar_ref, out_ref):
    b = pl.program_id(0)
    nc = pl.program_id(1)

    qrow = qbar_ref[0].astype(jnp.bfloat16).astype(jnp.float32)
    winv = 1.0 / jnp.clip(qrow, 1e-20, None)
    wsl = [jax.lax.slice(winv, (0, i), (1, i + 1)) for i in range(9)]

    m = mask_ref[0]

    shape = (_CH, _M)
    ni = jax.lax.broadcasted_iota(jnp.uint32, shape, 0)
    mi = jax.lax.broadcasted_iota(jnp.uint32, shape, 1)
    n0 = (nc * _N_CHUNK).astype(jnp.uint32)
    row0 = b.astype(jnp.uint32) * jnp.uint32(_N) + n0

    def bcast(row, col):
        return jax.lax.broadcast_in_dim(wsl[3 * row + col], shape, (0, 1))

    def chunk(i, carry):
        x = x0_ref[0, pl.ds(i * _CH, _CH), :]

        cell = (row0 + i.astype(jnp.uint32) * jnp.uint32(_CH) + ni) * jnp.uint32(_M) + mi
        j0 = cell * jnp.uint32(3)

        is1 = x == 1
        is2 = x == 2

        s = []
        for c in range(_NUM_CLASSES):
            e = _neglog_u_from_bits(_threefry_xored(j0 + jnp.uint32(c)))
            wc = jnp.where(is2, bcast(2, c), jnp.where(is1, bcast(1, c), bcast(0, c)))
            s.append(e * wc)

        idx = jnp.where(s[1] < s[0], 1, 0).astype(jnp.int32)
        sm = jnp.minimum(s[0], s[1])
        res = jnp.where(s[2] < sm, 2, idx).astype(jnp.int32)

        out_ref[0, pl.ds(i * _CH, _CH), :] = jnp.where(m != 0, res, 0)
        return carry

    jax.lax.fori_loop(0, _N_CHUNK // _CH, chunk, 0)


@jax.jit
def kernel(x_0, t, clause_mask, Q_bar_mats):
    x_0 = x_0.astype(jnp.int32)
    t = t.astype(jnp.int32)
    mask = clause_mask.astype(jnp.int32).reshape(_B, 1, _M)
    qbar = Q_bar_mats.astype(jnp.float32).reshape(1000, 1, 9)

    n_chunks = _N // _N_CHUNK
    grid = (_B, n_chunks)

    out = pl.pallas_call(
        _body,
        grid_spec=pltpu.PrefetchScalarGridSpec(
            num_scalar_prefetch=1,
            grid=grid,
            in_specs=[
                pl.BlockSpec((1, _N_CHUNK, _M), lambda b, nc, t_ref: (b, nc, 0)),
                pl.BlockSpec((1, 1, _M), lambda b, nc, t_ref: (b, 0, 0)),
                pl.BlockSpec((1, 1, 9), lambda b, nc, t_ref: (t_ref[b], 0, 0)),
            ],
            out_specs=pl.BlockSpec((1, _N_CHUNK, _M), lambda b, nc, t_ref: (b, nc, 0)),
        ),
        out_shape=jax.ShapeDtypeStruct((_B, _N, _M), jnp.int32),
        compiler_params=pltpu.CompilerParams(
            dimension_semantics=("parallel", "parallel"),
        ),
    )(t, x_0, mask, qbar)
    return out


# speedup vs baseline: 3.5550x; 1.0289x over previous
import jax
import jax.numpy as jnp
from jax.experimental import pallas as pl
from jax.experimental.pallas import tpu as pltpu

_NUM_CLASSES = 3
_B, _N, _M = 32, 256, 1024
_N_CHUNK = 128

_KS0 = 0
_KS1 = 12345
_KS2 = _KS0 ^ _KS1 ^ 0x1BD11BDA

_ROT0 = (13, 15, 26, 6)
_ROT1 = (17, 29, 16, 24)


def _rotl(x, d):
    return (x << jnp.uint32(d)) | (x >> jnp.uint32(32 - d))


def _four_rounds(x0, x1, rots):
    for r in rots:
        x0 = x0 + x1
        x1 = _rotl(x1, r)
        x1 = x1 ^ x0
    return x0, x1


def _threefry_xored(j):
    ks0 = jnp.uint32(_KS0)
    ks1 = jnp.uint32(_KS1)
    ks2 = jnp.uint32(_KS2)
    x1 = j + ks1
    x0 = x1
    x1 = _rotl(x1, _ROT0[0])
    x1 = x1 ^ x0
    for r in _ROT0[1:]:
        x0 = x0 + x1
        x1 = _rotl(x1, r)
        x1 = x1 ^ x0
    x0 = x0 + ks1
    x1 = x1 + (ks2 + jnp.uint32(1))
    x0, x1 = _four_rounds(x0, x1, _ROT1)
    x0 = x0 + ks2
    x1 = x1 + (ks0 + jnp.uint32(2))
    x0, x1 = _four_rounds(x0, x1, _ROT0)
    x0 = x0 + ks0
    x1 = x1 + (ks1 + jnp.uint32(3))
    x0, x1 = _four_rounds(x0, x1, _ROT1)
    x0 = x0 + ks1
    x1 = x1 + (ks2 + jnp.uint32(4))
    x0, x1 = _four_rounds(x0, x1, _ROT0)
    x0 = x0 + ks2
    x1 = x1 + (ks0 + jnp.uint32(5))
    return x0 ^ x1


def _neglog_u_from_bits(bits):
    tiny = jnp.float32(jnp.finfo(jnp.float32).tiny)
    one = jnp.float32(1.0)
    float_bits = (bits >> jnp.uint32(9)) | jnp.uint32(0x3F800000)
    floats = jax.lax.bitcast_convert_type(float_bits, jnp.float32) - one
    u = jnp.maximum(tiny, floats * (one - tiny) + tiny)
    return -jnp.log(u)


_CH = 16


def _body(t_ref, x0_ref, mask_ref, qbar_ref, out_ref):
    b = pl.program_id(0)
    nc = pl.program_id(1)

    qrow = qbar_ref[0].astype(jnp.bfloat16).astype(jnp.float32)
    winv = 1.0 / jnp.clip(qrow, 1e-20, None)
    wsl = [jax.lax.slice(winv, (0, i), (1, i + 1)) for i in range(9)]

    m = mask_ref[0]

    shape = (_CH, _M)
    ni = jax.lax.broadcasted_iota(jnp.uint32, shape, 0)
    mi = jax.lax.broadcasted_iota(jnp.uint32, shape, 1)
    n0 = (nc * _N_CHUNK).astype(jnp.uint32)
    row0 = b.astype(jnp.uint32) * jnp.uint32(_N) + n0

    def bcast(row, col):
        return jax.lax.broadcast_in_dim(wsl[3 * row + col], shape, (0, 1))

    j0_init = ((row0 + ni) * jnp.uint32(_M) + mi) * jnp.uint32(3)

    def chunk(i, j0):
        x = x0_ref[0, pl.ds(i * _CH, _CH), :]

        is1 = x == 1
        is2 = x == 2

        s = []
        for c in range(_NUM_CLASSES):
            e = _neglog_u_from_bits(_threefry_xored(j0 + jnp.uint32(c)))
            wc = jnp.where(is2, bcast(2, c), jnp.where(is1, bcast(1, c), bcast(0, c)))
            s.append(e * wc)

        idx = jnp.where(s[1] < s[0], 1, 0).astype(jnp.int32)
        sm = jnp.minimum(s[0], s[1])
        res = jnp.where(s[2] < sm, 2, idx).astype(jnp.int32)

        out_ref[0, pl.ds(i * _CH, _CH), :] = jnp.where(m != 0, res, 0)
        return j0 + jnp.uint32(3 * _CH * _M)

    jax.lax.fori_loop(0, _N_CHUNK // _CH, chunk, j0_init, unroll=8)


@jax.jit
def kernel(x_0, t, clause_mask, Q_bar_mats):
    x_0 = x_0.astype(jnp.int32)
    t = t.astype(jnp.int32)
    mask = clause_mask.astype(jnp.int32).reshape(_B, 1, _M)
    qbar = Q_bar_mats.astype(jnp.float32).reshape(1000, 1, 9)

    n_chunks = _N // _N_CHUNK
    grid = (_B, n_chunks)

    out = pl.pallas_call(
        _body,
        grid_spec=pltpu.PrefetchScalarGridSpec(
            num_scalar_prefetch=1,
            grid=grid,
            in_specs=[
                pl.BlockSpec((1, _N_CHUNK, _M), lambda b, nc, t_ref: (b, nc, 0)),
                pl.BlockSpec((1, 1, _M), lambda b, nc, t_ref: (b, 0, 0)),
                pl.BlockSpec((1, 1, 9), lambda b, nc, t_ref: (t_ref[b], 0, 0)),
            ],
            out_specs=pl.BlockSpec((1, _N_CHUNK, _M), lambda b, nc, t_ref: (b, nc, 0)),
        ),
        out_shape=jax.ShapeDtypeStruct((_B, _N, _M), jnp.int32),
        compiler_params=pltpu.CompilerParams(
            dimension_semantics=("parallel", "parallel"),
        ),
    )(t, x_0, mask, qbar)
    return out
